# baseline (speedup 1.0000x reference)
import jax
import jax.numpy as jnp
from jax import lax
from jax.experimental import pallas as pl
from jax.experimental.pallas import tpu as pltpu
from jax.experimental.pallas import tpu_sc as plsc

DIM = 32
BATCH = 16384
PACK = 4
PACKED_W = PACK * DIM
NUM_WORKERS = 32
B_PER_W = BATCH // NUM_WORKERS
CHUNK = 256
N_CHUNKS = B_PER_W // CHUNK
IDX_BLK = 128
BLK_PER_CHUNK = CHUNK // IDX_BLK
LANES = 16
GROUPS = CHUNK // LANES


def _mf_body(user_hbm, item_hbm, utab_hbm, itab_hbm, out_hbm,
             uidx_v, iidx_v, upid_v, ipid_v, urows_v, irows_v, out_v, sem):
    wid = lax.axis_index("s") * 2 + lax.axis_index("c")
    base = wid * B_PER_W

    iota = lax.iota(jnp.int32, LANES)

    for ch in range(N_CHUNKS):
        off = base + ch * CHUNK
        pltpu.sync_copy(user_hbm.at[pl.ds(off, CHUNK)], uidx_v)
        pltpu.sync_copy(item_hbm.at[pl.ds(off, CHUNK)], iidx_v)

        def pid_body(g, carry):
            us = uidx_v[pl.ds(g * LANES, LANES)]
            its = iidx_v[pl.ds(g * LANES, LANES)]
            blk = g // (IDX_BLK // LANES)
            lane0 = (g % (IDX_BLK // LANES)) * LANES
            upid_v[blk, pl.ds(lane0, LANES)] = lax.shift_right_logical(us, 2)
            ipid_v[blk, pl.ds(lane0, LANES)] = lax.shift_right_logical(its, 2)
            return carry

        lax.fori_loop(0, GROUPS, pid_body, 0)

        copies = []
        for j in range(BLK_PER_CHUNK):
            dst = pl.ds(j * IDX_BLK, IDX_BLK)
            copies.append(pltpu.async_copy(utab_hbm.at[upid_v.at[j]],
                                           urows_v.at[dst], sem))
            copies.append(pltpu.async_copy(itab_hbm.at[ipid_v.at[j]],
                                           irows_v.at[dst], sem))
        for c in copies:
            c.wait()

        def group_body(g, carry):
            rows = g * LANES + iota
            us = uidx_v[pl.ds(g * LANES, LANES)]
            its = iidx_v[pl.ds(g * LANES, LANES)]
            ucol0 = lax.shift_left(jnp.bitwise_and(us, 3), 5)
            icol0 = lax.shift_left(jnp.bitwise_and(its, 3), 5)
            acc = jnp.zeros((LANES,), jnp.float32)
            for d in range(DIM):
                uc = plsc.load_gather(urows_v, [rows, ucol0 + d])
                ic = plsc.load_gather(irows_v, [rows, icol0 + d])
                acc = acc + uc * ic
            out_v[pl.ds(ch * CHUNK + g * LANES, LANES)] = acc
            return carry

        lax.fori_loop(0, GROUPS, group_body, 0)

    pltpu.sync_copy(out_v, out_hbm.at[pl.ds(base, B_PER_W)])


@jax.jit
def _mf(user, item, user_table, item_table):
    utab = user_table.reshape(user_table.shape[0] // PACK, PACKED_W)
    itab = item_table.reshape(item_table.shape[0] // PACK, PACKED_W)
    mesh = plsc.VectorSubcoreMesh(core_axis_name="c", subcore_axis_name="s")
    return pl.kernel(
        _mf_body,
        out_type=jax.ShapeDtypeStruct((BATCH,), jnp.float32),
        mesh=mesh,
        compiler_params=pltpu.CompilerParams(
            needs_layout_passes=False,
            use_tc_tiling_on_sc=True,
        ),
        scratch_types=[
            pltpu.VMEM((CHUNK,), jnp.int32),
            pltpu.VMEM((CHUNK,), jnp.int32),
            pltpu.VMEM((BLK_PER_CHUNK, IDX_BLK), jnp.int32),
            pltpu.VMEM((BLK_PER_CHUNK, IDX_BLK), jnp.int32),
            pltpu.VMEM((CHUNK, PACKED_W), jnp.float32),
            pltpu.VMEM((CHUNK, PACKED_W), jnp.float32),
            pltpu.VMEM((B_PER_W,), jnp.float32),
            pltpu.SemaphoreType.DMA,
        ],
    )(user, item, utab, itab)


def kernel(user, item, user_table, item_table):
    return _mf(user, item, user_table, item_table)
